# Initial kernel scaffold; baseline (speedup 1.0000x reference)
import jax
import jax.numpy as jnp
from jax.experimental import pallas as pl


def kernel(data, decay, time, label, lab_mask, length, pid, W_lab, b_lab, W_inp, b_inp, W_med, b_med, We0, Wl0, Wi0, Wm0, We1, Wl1, Wi1, Wm1, Wq, Wk, Wv, Wo, W_beta, b_beta, W_out, b_out):
    raise NotImplementedError("write your pallas kernel here")


# speedup vs baseline: 2.6096x; 2.6096x over previous
import functools

import jax
import jax.numpy as jnp
from jax.experimental import pallas as pl
from jax.experimental.pallas import tpu as pltpu

LEN_LAB = 50
LEN_INP = 30
LEN_MED = 40
DIM_LAB = 64
DIM_INP = 32
DIM_MED = 32
DIM_EMBD = DIM_LAB + DIM_INP + DIM_MED
DIM_EMBD1 = 64
PL_ = 56
PI_ = 32
PM_ = 40

_NEG = -1e30


def _dot(a, b):
    return jax.lax.dot_general(
        a, b, (((1,), (0,)), ((), ())), preferred_element_type=jnp.float32
    )


def _dott(a, b):
    return jax.lax.dot_general(
        a, b, (((0,), (0,)), ((), ())), preferred_element_type=jnp.float32
    )


def _leaky(x):
    return jnp.where(x >= 0, x, 0.01 * x)


def _fwd_kernel(
    data_ref, decay_ref, mask_ref, len_ref,
    Wlab_ref, Winp_ref, Wmed_ref,
    We0_ref, Wl0_ref, Wi0_ref, Wm0_ref,
    We1_ref, Wl1_ref, Wi1_ref, Wm1_ref,
    Wq_ref, Wk_ref, Wv_ref, Wo_ref,
    Wbeta_ref, Wout_ref,
    ihl_ref, ihi_ref, ihm_ref,
    blab_ref, binp_ref, bmed_ref, bbeta_ref, bout_ref,
    out_ref,
    *, BB, T,
):
    BBT = BB * T
    f32 = jnp.float32

    d = data_ref[...]
    dec = decay_ref[...]
    msk = mask_ref[...]

    lab = d[:, :, 0:LEN_LAB]
    obs = (d[:, :, LEN_LAB:LEN_LAB + LEN_INP] != 0).astype(f32)
    med = d[:, :, LEN_LAB + LEN_INP:]

    z6 = jnp.zeros((BB, T, PL_ - LEN_LAB), f32)
    z2 = jnp.zeros((BB, T, PI_ - LEN_INP), f32)
    a_lp = jnp.concatenate([lab * dec, z6], axis=2)
    obsp = jnp.concatenate([obs, z2], axis=2)
    mskp = jnp.concatenate([msk, z6], axis=2)

    a_lp2 = a_lp.reshape(BBT, PL_)
    obsp2 = obsp.reshape(BBT, PI_)
    med2 = med.reshape(BBT, PM_)
    mskp2 = mskp.reshape(BBT, PL_)

    e_lab = _dot(mskp2, Wlab_ref[...]) + blab_ref[...]
    e_inp = _dot(obsp2, Winp_ref[...]) + binp_ref[...]
    e_med = _dot(med2, Wmed_ref[...]) + bmed_ref[...]

    We0 = We0_ref[...]
    he0 = (
        _dot(e_lab, We0[0:DIM_LAB])
        + _dot(e_inp, We0[DIM_LAB:DIM_LAB + DIM_INP])
        + _dot(e_med, We0[DIM_LAB + DIM_INP:])
    )

    ihl = ihl_ref[...]
    ihi = ihi_ref[...]
    ihm = ihm_ref[...]

    Nw_l0 = _dot(ihl, Wl0_ref[...])
    Nw_i0 = _dot(ihi, Wi0_ref[...])
    Nw_m0 = _dot(ihm, Wm0_ref[...])
    g0 = he0 + _dot(a_lp2, Nw_l0) + _dot(obsp2, Nw_i0) + _dot(med2, Nw_m0)

    he0_3 = he0.reshape(BB, T, DIM_EMBD)
    hl1 = jnp.stack([ihl + _dott(a_lp[b], he0_3[b]) for b in range(BB)])
    hi1 = jnp.stack([ihi + _dott(obsp[b], he0_3[b]) for b in range(BB)])
    hm1 = jnp.stack([ihm + _dott(med[b], he0_3[b]) for b in range(BB)])

    h_e1 = _leaky(g0)
    hl1 = _leaky(hl1)
    hi1 = _leaky(hi1)
    hm1 = _leaky(hm1)

    he1 = _dot(h_e1, We1_ref[...])
    Nw_l1 = _dot(hl1.reshape(BB * PL_, DIM_EMBD), Wl1_ref[...]).reshape(BB, PL_, DIM_EMBD)
    Nw_i1 = _dot(hi1.reshape(BB * PI_, DIM_EMBD), Wi1_ref[...]).reshape(BB, PI_, DIM_EMBD)
    Nw_m1 = _dot(hm1.reshape(BB * PM_, DIM_EMBD), Wm1_ref[...]).reshape(BB, PM_, DIM_EMBD)

    he1_3 = he1.reshape(BB, T, DIM_EMBD)
    hl2 = hl1 + jnp.stack([_dott(a_lp[b], he1_3[b]) for b in range(BB)])
    hi2 = hi1 + jnp.stack([_dott(obsp[b], he1_3[b]) for b in range(BB)])
    hm2 = hm1 + jnp.stack([_dott(med[b], he1_3[b]) for b in range(BB)])

    lc = jnp.clip(len_ref[...], 1.0, float(T))
    tstar = (lc - 1.0).astype(jnp.int32)
    iota_t = jax.lax.broadcasted_iota(jnp.int32, (BB, T), 1)
    onehot = (iota_t == tstar).astype(f32)
    oh = onehot[:, :, None]

    he_star = jnp.sum(he1_3 * oh, axis=1)
    al_star = jnp.sum(a_lp * oh, axis=1)
    ob_star = jnp.sum(obsp * oh, axis=1)
    md_star = jnp.sum(med * oh, axis=1)

    q_in = (
        he_star
        + jnp.sum(al_star[:, :, None] * Nw_l1, axis=1)
        + jnp.sum(ob_star[:, :, None] * Nw_i1, axis=1)
        + jnp.sum(md_star[:, :, None] * Nw_m1, axis=1)
    )

    q = _dot(q_in, Wq_ref[...])
    Wk = Wk_ref[...]
    Wv = Wv_ref[...]
    Kl = _dot(hl2.reshape(BB * PL_, DIM_EMBD), Wk).reshape(BB, PL_, DIM_EMBD)
    Ki = _dot(hi2.reshape(BB * PI_, DIM_EMBD), Wk).reshape(BB, PI_, DIM_EMBD)
    Km = _dot(hm2.reshape(BB * PM_, DIM_EMBD), Wk).reshape(BB, PM_, DIM_EMBD)
    Vl = _dot(hl2.reshape(BB * PL_, DIM_EMBD), Wv).reshape(BB, PL_, DIM_EMBD)
    Vi = _dot(hi2.reshape(BB * PI_, DIM_EMBD), Wv).reshape(BB, PI_, DIM_EMBD)
    Vm = _dot(hm2.reshape(BB * PM_, DIM_EMBD), Wv).reshape(BB, PM_, DIM_EMBD)

    scale = 1.0 / (float(DIM_EMBD) ** 0.5)
    qn = q[:, None, :]
    sl = jnp.sum(qn * Kl, axis=2) * scale
    si = jnp.sum(qn * Ki, axis=2) * scale
    sm = jnp.sum(qn * Km, axis=2) * scale

    il = jax.lax.broadcasted_iota(jnp.int32, (BB, PL_), 1)
    ii = jax.lax.broadcasted_iota(jnp.int32, (BB, PI_), 1)
    sl = jnp.where(il < LEN_LAB, sl, _NEG)
    si = jnp.where(ii < LEN_INP, si, _NEG)

    mx = jnp.maximum(
        jnp.max(sl, axis=1, keepdims=True),
        jnp.maximum(jnp.max(si, axis=1, keepdims=True),
                    jnp.max(sm, axis=1, keepdims=True)),
    )
    pl_ = jnp.exp(sl - mx)
    pi_ = jnp.exp(si - mx)
    pm_ = jnp.exp(sm - mx)
    zden = (
        jnp.sum(pl_, axis=1, keepdims=True)
        + jnp.sum(pi_, axis=1, keepdims=True)
        + jnp.sum(pm_, axis=1, keepdims=True)
    )
    ctx = (
        jnp.sum(pl_[:, :, None] * Vl, axis=1)
        + jnp.sum(pi_[:, :, None] * Vi, axis=1)
        + jnp.sum(pm_[:, :, None] * Vm, axis=1)
    ) / zden

    h_out = _dot(ctx + q_in, Wo_ref[...])
    beta = jnp.tanh(_dot(h_out, Wbeta_ref[...]) + bbeta_ref[...])
    s = _dot(beta, Wout_ref[...]) + bout_ref[...]
    s0 = s[:, 0:1]
    s1 = s[:, 1:2]
    p0 = jax.nn.sigmoid(s0 - s1)
    p1 = jax.nn.sigmoid(s1 - s0)
    lane = jax.lax.broadcasted_iota(jnp.int32, (BB, DIM_EMBD), 1)
    out_ref[...] = jnp.where(lane == 0, p0, jnp.where(lane == 1, p1, 0.0))


def _run(data, decay, lab_mask, lenf,
         Wlab_p, Winp_p, W_med,
         We0, Wl0, Wi0, Wm0, We1, Wl1, Wi1, Wm1,
         Wq, Wk, Wv, Wo, W_beta, Wout_p,
         ihl, ihi, ihm, blab, binp, bmed, bbeta, bout_p):
    B, T, _ = data.shape
    BB = 8
    grid = (B // BB,)

    def blk(i):
        return (i, 0, 0)

    def blk2(i):
        return (i, 0)

    def full2(i):
        return (0, 0)

    w_specs = [
        pl.BlockSpec(w.shape, full2)
        for w in (Wlab_p, Winp_p, W_med, We0, Wl0, Wi0, Wm0, We1, Wl1, Wi1,
                  Wm1, Wq, Wk, Wv, Wo, W_beta, Wout_p, ihl, ihi, ihm,
                  blab, binp, bmed, bbeta, bout_p)
    ]

    out = pl.pallas_call(
        functools.partial(_fwd_kernel, BB=BB, T=T),
        grid=grid,
        in_specs=[
            pl.BlockSpec((BB, T, data.shape[2]), blk),
            pl.BlockSpec((BB, T, LEN_LAB), blk),
            pl.BlockSpec((BB, T, LEN_LAB), blk),
            pl.BlockSpec((BB, 1), blk2),
        ] + w_specs,
        out_specs=pl.BlockSpec((BB, DIM_EMBD), blk2),
        out_shape=jax.ShapeDtypeStruct((B, DIM_EMBD), jnp.float32),
        compiler_params=pltpu.CompilerParams(
            dimension_semantics=("arbitrary",),
        ),
    )(data, decay, lab_mask, lenf,
      Wlab_p, Winp_p, W_med,
      We0, Wl0, Wi0, Wm0, We1, Wl1, Wi1, Wm1,
      Wq, Wk, Wv, Wo, W_beta, Wout_p,
      ihl, ihi, ihm, blab, binp, bmed, bbeta, bout_p)
    return out[:, :2]


def kernel(data, decay, time, label, lab_mask, length, pid,
           W_lab, b_lab, W_inp, b_inp, W_med, b_med,
           We0, Wl0, Wi0, Wm0, We1, Wl1, Wi1, Wm1,
           Wq, Wk, Wv, Wo, W_beta, b_beta, W_out, b_out):
    f32 = jnp.float32
    B = data.shape[0]

    Wlab_p = jnp.zeros((PL_, DIM_LAB), f32).at[:LEN_LAB].set(W_lab)
    Winp_p = jnp.zeros((PI_, DIM_INP), f32).at[:LEN_INP].set(W_inp)

    ihl = jnp.zeros((PL_, DIM_EMBD), f32).at[:LEN_LAB, :DIM_LAB].set(W_lab + b_lab[None, :])
    ihi = jnp.zeros((PI_, DIM_EMBD), f32).at[:LEN_INP, DIM_LAB:DIM_LAB + DIM_INP].set(W_inp + b_inp[None, :])
    ihm = jnp.zeros((PM_, DIM_EMBD), f32).at[:LEN_MED, DIM_LAB + DIM_INP:].set(W_med + b_med[None, :])

    Wout_p = jnp.zeros((DIM_EMBD1, DIM_EMBD), f32).at[:, :2].set(W_out)
    bout_p = jnp.zeros((1, DIM_EMBD), f32).at[0, :2].set(b_out)

    lenf = length.astype(f32).reshape(B, 1)

    logit = _run(
        data, decay, lab_mask, lenf,
        Wlab_p, Winp_p, W_med,
        We0, Wl0, Wi0, Wm0, We1, Wl1, Wi1, Wm1,
        Wq, Wk, Wv, Wo, W_beta, Wout_p,
        ihl, ihi, ihm,
        b_lab.reshape(1, -1), b_inp.reshape(1, -1), b_med.reshape(1, -1),
        b_beta.reshape(1, -1), bout_p,
    )
    return (logit, label)


# speedup vs baseline: 3.1177x; 1.1947x over previous
import functools

import jax
import jax.numpy as jnp
from jax.experimental import pallas as pl
from jax.experimental.pallas import tpu as pltpu

LEN_LAB = 50
LEN_INP = 30
LEN_MED = 40
DIM_LAB = 64
DIM_INP = 32
DIM_MED = 32
DIM_EMBD = DIM_LAB + DIM_INP + DIM_MED
DIM_EMBD1 = 64
OL = 0
OI = 56
OM = 88
ND = 128

_NEG = -1e30


def _dot(a, b):
    return jax.lax.dot_general(
        a, b, (((1,), (0,)), ((), ())), preferred_element_type=jnp.float32
    )


def _dott(a, b):
    return jax.lax.dot_general(
        a, b, (((0,), (0,)), ((), ())), preferred_element_type=jnp.float32
    )


def _leaky(x):
    return jnp.maximum(x, 0.01 * x)


def _fwd_kernel(
    data_ref, decay_ref, mask_ref, len_ref,
    We0f_ref, be0_ref, Nw0_ref, ih_ref,
    We1_ref, Wl1_ref, Wi1_ref, Wm1_ref,
    Wq_ref, WkT_ref, Wv_ref, Wo_ref,
    Wbeta_ref, bbeta_ref, Wout_ref, bout_ref,
    out_ref,
    N2_s, qin_s,
    *, BB, T,
):
    BBT = BB * T
    f32 = jnp.float32
    bf16 = jnp.bfloat16

    N2p = N2_s[...]
    q_inp = qin_s[...]
    q = _dot(q_inp, Wq_ref[...])
    qk = _dot(q, WkT_ref[...])
    scale = 1.0 / (float(DIM_EMBD) ** 0.5)
    s = jnp.sum(qk[:, None, :] * N2p, axis=2) * scale
    nn = jax.lax.broadcasted_iota(jnp.int32, (BB, ND), 1)
    valid = (nn < LEN_LAB) | ((nn >= OI) & (nn < OI + LEN_INP)) | (nn >= OM)
    s = jnp.where(valid, s, _NEG)
    mx = jnp.max(s, axis=1, keepdims=True)
    p = jnp.exp(s - mx)
    p = p / jnp.sum(p, axis=1, keepdims=True)
    u = jnp.sum(p[:, :, None] * N2p, axis=1)
    ctx = _dot(u, Wv_ref[...])
    h_out = _dot(ctx + q_inp, Wo_ref[...])
    beta = jnp.tanh(_dot(h_out, Wbeta_ref[...]) + bbeta_ref[...])
    sc = _dot(beta, Wout_ref[...]) + bout_ref[...]
    s0 = sc[:, 0:1]
    s1 = sc[:, 1:2]
    p0 = jax.nn.sigmoid(s0 - s1)
    p1 = jax.nn.sigmoid(s1 - s0)
    lane = jax.lax.broadcasted_iota(jnp.int32, (BB, DIM_EMBD), 1)
    out_ref[...] = jnp.where(lane == 0, p0, jnp.where(lane == 1, p1, 0.0))

    d = data_ref[...]
    dec = decay_ref[...]
    msk = mask_ref[...]

    lab = d[:, :, 0:LEN_LAB]
    obs = (d[:, :, LEN_LAB:LEN_LAB + LEN_INP] != 0).astype(f32)
    med = d[:, :, LEN_LAB + LEN_INP:]

    z6 = jnp.zeros((BB, T, OI - LEN_LAB), f32)
    z2 = jnp.zeros((BB, T, OM - OI - LEN_INP), f32)
    tail = jnp.concatenate([obs, z2, med], axis=2)
    X = jnp.concatenate([msk, z6, tail], axis=2).reshape(BBT, ND)
    A = jnp.concatenate([lab * dec, z6, tail], axis=2)
    A2 = A.reshape(BBT, ND)

    he0 = _dot(X, We0f_ref[...]) + be0_ref[...]
    g0 = he0 + _dot(A2, Nw0_ref[...])

    Ab = A.astype(bf16)
    he0b = he0.astype(bf16).reshape(BB, T, DIM_EMBD)
    ih = ih_ref[...]
    N1 = jnp.stack([ih + _dott(Ab[b], he0b[b]) for b in range(BB)])

    h_e1 = _leaky(g0)
    N1 = _leaky(N1)

    he1 = _dot(h_e1, We1_ref[...])
    he1b = he1.astype(bf16).reshape(BB, T, DIM_EMBD)
    N2 = N1 + jnp.stack([_dott(Ab[b], he1b[b]) for b in range(BB)])

    lc = jnp.clip(len_ref[...], 1.0, float(T))
    tstar = (lc - 1.0).astype(jnp.int32)
    iota_t = jax.lax.broadcasted_iota(jnp.int32, (BB, T), 1)
    onehot = (iota_t == tstar).astype(f32)
    oh = onehot[:, :, None]

    he1_3 = he1.reshape(BB, T, DIM_EMBD)
    he_star = jnp.sum(he1_3 * oh, axis=1)
    a_star = jnp.sum(A * oh, axis=1)

    wl = jnp.sum(a_star[:, OL:OI, None] * N1[:, OL:OI, :], axis=1)
    wi = jnp.sum(a_star[:, OI:OM, None] * N1[:, OI:OM, :], axis=1)
    wm = jnp.sum(a_star[:, OM:ND, None] * N1[:, OM:ND, :], axis=1)
    q_in = (he_star + _dot(wl, Wl1_ref[...]) + _dot(wi, Wi1_ref[...])
            + _dot(wm, Wm1_ref[...]))

    N2_s[...] = N2
    qin_s[...] = q_in


def _run(data, decay, lab_mask, lenf, consts):
    B, T, F = data.shape
    BB = 8
    G = B // BB
    grid = (G + 1,)

    def blk(i):
        j = jnp.minimum(i, G - 1)
        return (j, 0, 0)

    def blk2(i):
        j = jnp.minimum(i, G - 1)
        return (j, 0)

    def out_blk(i):
        return (jnp.maximum(i - 1, 0), 0)

    def full2(i):
        return (0, 0)

    w_specs = [pl.BlockSpec(w.shape, full2) for w in consts]

    out = pl.pallas_call(
        functools.partial(_fwd_kernel, BB=BB, T=T),
        grid=grid,
        in_specs=[
            pl.BlockSpec((BB, T, F), blk),
            pl.BlockSpec((BB, T, LEN_LAB), blk),
            pl.BlockSpec((BB, T, LEN_LAB), blk),
            pl.BlockSpec((BB, 1), blk2),
        ] + w_specs,
        out_specs=pl.BlockSpec((BB, DIM_EMBD), out_blk),
        out_shape=jax.ShapeDtypeStruct((B, DIM_EMBD), jnp.float32),
        scratch_shapes=[
            pltpu.VMEM((BB, ND, DIM_EMBD), jnp.float32),
            pltpu.VMEM((BB, DIM_EMBD), jnp.float32),
        ],
        compiler_params=pltpu.CompilerParams(
            dimension_semantics=("arbitrary",),
        ),
    )(data, decay, lab_mask, lenf, *consts)
    return out[:, :2]


def kernel(data, decay, time, label, lab_mask, length, pid,
           W_lab, b_lab, W_inp, b_inp, W_med, b_med,
           We0, Wl0, Wi0, Wm0, We1, Wl1, Wi1, Wm1,
           Wq, Wk, Wv, Wo, W_beta, b_beta, W_out, b_out):
    f32 = jnp.float32
    B = data.shape[0]
    hp = jax.lax.Precision.HIGHEST

    Wemb = jnp.zeros((ND, DIM_EMBD), f32)
    Wemb = Wemb.at[OL:OL + LEN_LAB, 0:DIM_LAB].set(W_lab)
    Wemb = Wemb.at[OI:OI + LEN_INP, DIM_LAB:DIM_LAB + DIM_INP].set(W_inp)
    Wemb = Wemb.at[OM:OM + LEN_MED, DIM_LAB + DIM_INP:].set(W_med)
    bcat = jnp.concatenate([b_lab, b_inp, b_med])[None, :]
    We0f = jnp.dot(Wemb, We0, precision=hp)
    be0 = jnp.dot(bcat, We0, precision=hp)

    ih = jnp.zeros((ND, DIM_EMBD), f32)
    ih = ih.at[OL:OL + LEN_LAB, 0:DIM_LAB].set(W_lab + b_lab[None, :])
    ih = ih.at[OI:OI + LEN_INP, DIM_LAB:DIM_LAB + DIM_INP].set(W_inp + b_inp[None, :])
    ih = ih.at[OM:OM + LEN_MED, DIM_LAB + DIM_INP:].set(W_med + b_med[None, :])
    Nw0 = jnp.concatenate([
        jnp.dot(ih[0:OI], Wl0, precision=hp),
        jnp.dot(ih[OI:OM], Wi0, precision=hp),
        jnp.dot(ih[OM:ND], Wm0, precision=hp),
    ], axis=0)

    Wout_p = jnp.zeros((DIM_EMBD1, DIM_EMBD), f32).at[:, :2].set(W_out)
    bout_p = jnp.zeros((1, DIM_EMBD), f32).at[0, :2].set(b_out)

    lenf = length.astype(f32).reshape(B, 1)

    consts = (We0f, be0, Nw0, ih,
              We1, Wl1, Wi1, Wm1, Wq, Wk.T, Wv, Wo,
              W_beta, b_beta.reshape(1, -1), Wout_p, bout_p)

    logit = _run(data, decay, lab_mask, lenf, consts)
    return (logit, label)


# speedup vs baseline: 3.2128x; 1.0305x over previous
import functools

import jax
import jax.numpy as jnp
from jax.experimental import pallas as pl
from jax.experimental.pallas import tpu as pltpu

LEN_LAB = 50
LEN_INP = 30
LEN_MED = 40
DIM_LAB = 64
DIM_INP = 32
DIM_MED = 32
DIM_EMBD = DIM_LAB + DIM_INP + DIM_MED
DIM_EMBD1 = 64
OL = 0
OI = 56
OM = 88
ND = 128

_NEG = -1e30


def _dot(a, b):
    return jax.lax.dot_general(
        a, b, (((1,), (0,)), ((), ())), preferred_element_type=jnp.float32
    )


def _dott(a, b):
    return jax.lax.dot_general(
        a, b, (((0,), (0,)), ((), ())), preferred_element_type=jnp.float32
    )


def _leaky(x):
    return jnp.maximum(x, 0.01 * x)


def _fwd_kernel(
    data_ref, decay_ref, mask_ref, len_ref,
    We0f_ref, be0_ref, Nw0_ref, ih_ref,
    We1_ref, Wl1_ref, Wi1_ref, Wm1_ref,
    Wqk_ref, WvoB_ref, WoB_ref,
    bbeta_ref, Wout_ref, bout_ref,
    out_ref,
    N2_s, qk_s, r_s,
    *, BB, T,
):
    BBT = BB * T
    f32 = jnp.float32
    bf16 = jnp.bfloat16

    N2p = N2_s[...]
    qk = qk_s[...]
    scale = 1.0 / (float(DIM_EMBD) ** 0.5)
    s = jnp.sum(qk[:, None, :] * N2p, axis=2) * scale
    nn = jax.lax.broadcasted_iota(jnp.int32, (BB, ND), 1)
    valid = (nn < LEN_LAB) | ((nn >= OI) & (nn < OI + LEN_INP)) | (nn >= OM)
    s = jnp.where(valid, s, _NEG)
    mx = jnp.max(s, axis=1, keepdims=True)
    p = jnp.exp(s - mx)
    p = p / jnp.sum(p, axis=1, keepdims=True)
    u = jnp.sum(p[:, :, None] * N2p, axis=1)
    beta = jnp.tanh(_dot(u, WvoB_ref[...]) + r_s[...] + bbeta_ref[...])
    sc = _dot(beta, Wout_ref[...]) + bout_ref[...]
    s0 = sc[:, 0:1]
    s1 = sc[:, 1:2]
    p0 = jax.nn.sigmoid(s0 - s1)
    p1 = jax.nn.sigmoid(s1 - s0)
    lane = jax.lax.broadcasted_iota(jnp.int32, (BB, DIM_EMBD), 1)
    out_ref[...] = jnp.where(lane == 0, p0, jnp.where(lane == 1, p1, 0.0))

    d = data_ref[...]
    dec = decay_ref[...]
    msk = mask_ref[...]

    lab = d[:, :, 0:LEN_LAB]
    obs = (d[:, :, LEN_LAB:LEN_LAB + LEN_INP] != 0).astype(f32)
    med = d[:, :, LEN_LAB + LEN_INP:]

    z6 = jnp.zeros((BB, T, OI - LEN_LAB), f32)
    z2 = jnp.zeros((BB, T, OM - OI - LEN_INP), f32)
    tail = jnp.concatenate([obs, z2, med], axis=2)
    X = jnp.concatenate([msk, z6, tail], axis=2).reshape(BBT, ND)
    A = jnp.concatenate([lab * dec, z6, tail], axis=2)
    A2 = A.reshape(BBT, ND)

    he0 = _dot(X, We0f_ref[...]) + be0_ref[...]
    g0 = he0 + _dot(A2, Nw0_ref[...])

    Ab = A.astype(bf16)
    he0b = he0.astype(bf16).reshape(BB, T, DIM_EMBD)
    ih = ih_ref[...]
    N1 = jnp.stack([ih + _dott(Ab[b], he0b[b]) for b in range(BB)])

    h_e1 = _leaky(g0)
    N1 = _leaky(N1)

    he1 = _dot(h_e1, We1_ref[...])
    he1b = he1.astype(bf16).reshape(BB, T, DIM_EMBD)
    N2 = N1 + jnp.stack([_dott(Ab[b], he1b[b]) for b in range(BB)])

    lc = jnp.clip(len_ref[...], 1.0, float(T))
    tstar = (lc - 1.0).astype(jnp.int32)
    iota_t = jax.lax.broadcasted_iota(jnp.int32, (BB, T), 1)
    onehot = (iota_t == tstar).astype(f32)
    oh = onehot[:, :, None]

    he1_3 = he1.reshape(BB, T, DIM_EMBD)
    he_star = jnp.sum(he1_3 * oh, axis=1)
    a_star = jnp.sum(A * oh, axis=1)

    wl = jnp.sum(a_star[:, OL:OI, None] * N1[:, OL:OI, :], axis=1)
    wi = jnp.sum(a_star[:, OI:OM, None] * N1[:, OI:OM, :], axis=1)
    wm = jnp.sum(a_star[:, OM:ND, None] * N1[:, OM:ND, :], axis=1)
    q_in = (he_star + _dot(wl, Wl1_ref[...]) + _dot(wi, Wi1_ref[...])
            + _dot(wm, Wm1_ref[...]))

    N2_s[...] = N2
    qk_s[...] = _dot(q_in, Wqk_ref[...])
    r_s[...] = _dot(q_in, WoB_ref[...])


def _run(data, decay, lab_mask, lenf, consts):
    B, T, F = data.shape
    BB = 8
    G = B // BB
    grid = (G + 1,)

    def blk(i):
        j = jnp.minimum(i, G - 1)
        return (j, 0, 0)

    def blk2(i):
        j = jnp.minimum(i, G - 1)
        return (j, 0)

    def out_blk(i):
        return (jnp.maximum(i - 1, 0), 0)

    def full2(i):
        return (0, 0)

    w_specs = [pl.BlockSpec(w.shape, full2) for w in consts]

    out = pl.pallas_call(
        functools.partial(_fwd_kernel, BB=BB, T=T),
        grid=grid,
        in_specs=[
            pl.BlockSpec((BB, T, F), blk),
            pl.BlockSpec((BB, T, LEN_LAB), blk),
            pl.BlockSpec((BB, T, LEN_LAB), blk),
            pl.BlockSpec((BB, 1), blk2),
        ] + w_specs,
        out_specs=pl.BlockSpec((BB, DIM_EMBD), out_blk),
        out_shape=jax.ShapeDtypeStruct((B, DIM_EMBD), jnp.float32),
        scratch_shapes=[
            pltpu.VMEM((BB, ND, DIM_EMBD), jnp.float32),
            pltpu.VMEM((BB, DIM_EMBD), jnp.float32),
            pltpu.VMEM((BB, DIM_EMBD1), jnp.float32),
        ],
        compiler_params=pltpu.CompilerParams(
            dimension_semantics=("arbitrary",),
        ),
    )(data, decay, lab_mask, lenf, *consts)
    return out[:, :2]


def kernel(data, decay, time, label, lab_mask, length, pid,
           W_lab, b_lab, W_inp, b_inp, W_med, b_med,
           We0, Wl0, Wi0, Wm0, We1, Wl1, Wi1, Wm1,
           Wq, Wk, Wv, Wo, W_beta, b_beta, W_out, b_out):
    f32 = jnp.float32
    B = data.shape[0]
    hp = jax.lax.Precision.HIGHEST

    Wemb = jnp.zeros((ND, DIM_EMBD), f32)
    Wemb = Wemb.at[OL:OL + LEN_LAB, 0:DIM_LAB].set(W_lab)
    Wemb = Wemb.at[OI:OI + LEN_INP, DIM_LAB:DIM_LAB + DIM_INP].set(W_inp)
    Wemb = Wemb.at[OM:OM + LEN_MED, DIM_LAB + DIM_INP:].set(W_med)
    bcat = jnp.concatenate([b_lab, b_inp, b_med])[None, :]
    We0f = jnp.dot(Wemb, We0, precision=hp)
    be0 = jnp.dot(bcat, We0, precision=hp)

    ih = jnp.zeros((ND, DIM_EMBD), f32)
    ih = ih.at[OL:OL + LEN_LAB, 0:DIM_LAB].set(W_lab + b_lab[None, :])
    ih = ih.at[OI:OI + LEN_INP, DIM_LAB:DIM_LAB + DIM_INP].set(W_inp + b_inp[None, :])
    ih = ih.at[OM:OM + LEN_MED, DIM_LAB + DIM_INP:].set(W_med + b_med[None, :])
    Nw0 = jnp.concatenate([
        jnp.dot(ih[0:OI], Wl0, precision=hp),
        jnp.dot(ih[OI:OM], Wi0, precision=hp),
        jnp.dot(ih[OM:ND], Wm0, precision=hp),
    ], axis=0)

    Wout_p = jnp.zeros((DIM_EMBD1, DIM_EMBD), f32).at[:, :2].set(W_out)
    bout_p = jnp.zeros((1, DIM_EMBD), f32).at[0, :2].set(b_out)

    Wqk = jnp.dot(Wq, Wk.T, precision=hp)
    WoB = jnp.dot(Wo, W_beta, precision=hp)
    WvoB = jnp.dot(Wv, WoB, precision=hp)

    lenf = length.astype(f32).reshape(B, 1)

    consts = (We0f, be0, Nw0, ih,
              We1, Wl1, Wi1, Wm1, Wqk, WvoB, WoB,
              b_beta.reshape(1, -1), Wout_p, bout_p)

    logit = _run(data, decay, lab_mask, lenf, consts)
    return (logit, label)
